# Initial kernel scaffold; baseline (speedup 1.0000x reference)
import jax
import jax.numpy as jnp
from jax.experimental import pallas as pl


def kernel(classifications, regressions, anchors, annotations, criterion, transcription, selected_indices, probs_sizes, pool_w, htr_gt_box):
    raise NotImplementedError("write your pallas kernel here")


# speedup vs baseline: 4.1671x; 4.1671x over previous
import functools

import jax
import jax.numpy as jnp
from jax import lax
from jax.experimental import pallas as pl
from jax.experimental.pallas import tpu as pltpu
from jax.experimental.pallas import tpu_sc as plsc

_NC = 2
_NS = 16
_NW = _NC * _NS
_LANES = 16

_ALPHA = 0.25
_AB = 4096


def _sc_match(anc_t, b4p, bb, *, apad, gp, g_real, n_batch):
    B = n_batch
    tiles_per_batch = _NW // B
    ch = apad // tiles_per_batch
    nv = ch // _LANES

    mesh = plsc.VectorSubcoreMesh(core_axis_name="c", subcore_axis_name="s")

    def body(anc_hbm, b4_hbm, bb_hbm, out_hbm, a_v, b4_v, bb_v, o_v):
        wid = lax.axis_index("s") * _NC + lax.axis_index("c")
        j = wid // tiles_per_batch
        q = wid % tiles_per_batch
        base = q * ch
        for c in range(4):
            pltpu.sync_copy(anc_hbm.at[pl.ds(c * apad + base, ch)],
                            a_v.at[pl.ds(c * ch, ch)])
        pltpu.sync_copy(b4_hbm.at[pl.ds(j * gp * 4, gp * 4)], b4_v)
        pltpu.sync_copy(bb_hbm.at[pl.ds(j * gp * 5 * _LANES, gp * 5 * _LANES)],
                        bb_v)

        def outer(v, _):
            s = v * _LANES
            ax1 = a_v[pl.ds(s, _LANES)]
            ay1 = a_v[pl.ds(ch + s, _LANES)]
            ax2 = a_v[pl.ds(2 * ch + s, _LANES)]
            ay2 = a_v[pl.ds(3 * ch + s, _LANES)]
            aw = ax2 - ax1
            ah = ay2 - ay1
            aarea = aw * ah
            cur_max = jnp.full((_LANES,), -1.0, jnp.float32)
            cur_idx = jnp.zeros((_LANES,), jnp.int32)
            for g in range(g_real):
                gb = g * 5 * _LANES
                bx1 = bb_v[pl.ds(gb, _LANES)]
                by1 = bb_v[pl.ds(gb + _LANES, _LANES)]
                bx2 = bb_v[pl.ds(gb + 2 * _LANES, _LANES)]
                by2 = bb_v[pl.ds(gb + 3 * _LANES, _LANES)]
                ba = bb_v[pl.ds(gb + 4 * _LANES, _LANES)]
                iw = jnp.minimum(ax2, bx2) - jnp.maximum(ax1, bx1)
                iw = jnp.maximum(iw, 0.0)
                ih = jnp.minimum(ay2, by2) - jnp.maximum(ay1, by1)
                ih = jnp.maximum(ih, 0.0)
                inter = iw * ih
                ua = jnp.maximum(aarea + ba - inter, 1e-8)
                iou = inter / ua
                upd = iou > cur_max
                cur_max = jnp.where(upd, iou, cur_max)
                cur_idx = jnp.where(upd, jnp.int32(g), cur_idx)
            idx4 = cur_idx * 4
            gx1 = plsc.load_gather(b4_v, [idx4])
            gy1 = plsc.load_gather(b4_v, [idx4 + 1])
            gx2 = plsc.load_gather(b4_v, [idx4 + 2])
            gy2 = plsc.load_gather(b4_v, [idx4 + 3])
            gw = gx2 - gx1
            gh = gy2 - gy1
            gcx = gx1 + 0.5 * gw
            gcy = gy1 + 0.5 * gh
            gw = jnp.maximum(gw, 1.0)
            gh = jnp.maximum(gh, 1.0)
            acx = ax1 + 0.5 * aw
            acy = ay1 + 0.5 * ah
            o_v[pl.ds(s, _LANES)] = cur_max
            o_v[pl.ds(ch + s, _LANES)] = (gcx - acx) / aw
            o_v[pl.ds(2 * ch + s, _LANES)] = (gcy - acy) / ah
            o_v[pl.ds(3 * ch + s, _LANES)] = gw / aw
            o_v[pl.ds(4 * ch + s, _LANES)] = gh / ah
            return 0

        lax.fori_loop(0, nv, outer, 0)
        for r in range(5):
            pltpu.sync_copy(
                o_v.at[pl.ds(r * ch, ch)],
                out_hbm.at[pl.ds(r * B * apad + j * apad + base, ch)])

    return pl.kernel(
        body,
        out_type=jax.ShapeDtypeStruct((5 * B * apad,), jnp.float32),
        mesh=mesh,
        compiler_params=pltpu.CompilerParams(needs_layout_passes=False),
        scratch_types=[
            pltpu.VMEM((4 * ch,), jnp.float32),
            pltpu.VMEM((gp * 4,), jnp.float32),
            pltpu.VMEM((gp * 5 * _LANES,), jnp.float32),
            pltpu.VMEM((5 * ch,), jnp.float32),
        ],
    )(anc_t, b4p, bb)


def _tc_body(cls_ref, p0_ref, reg_ref, match_ref, out_ref, *, a_real, nblk):
    i = pl.program_id(1)
    x = cls_ref[0]
    p = jnp.where(x < 1.0 - 1e-4, x, 1.0 - 1e-4)
    p = jnp.where(p > 1e-4, p, 1e-4)
    l0mat = (p * p) * (-jnp.log(1.0 - p))

    m = match_ref[:, 0, 0, :]
    iou = m[0]
    pos = iou >= 0.5
    neg = iou < 0.4
    gid = i * _AB + lax.iota(jnp.int32, _AB)
    validm = gid < a_real
    posmask = validm & pos
    w1f = jnp.where(validm & (pos | neg), 1.0, 0.0).reshape(1, _AB)

    term1 = (1.0 - _ALPHA) * jnp.sum(
        jnp.dot(w1f, l0mat, preferred_element_type=jnp.float32))

    p0 = jnp.clip(p0_ref[0, 0, :], 1e-4, 1.0 - 1e-4)
    l0r = (1.0 - _ALPHA) * (p0 * p0) * (-jnp.log(1.0 - p0))
    l1r = _ALPHA * ((1.0 - p0) * (1.0 - p0)) * (-jnp.log(p0))
    corr = jnp.sum(jnp.where(posmask, l1r - l0r, 0.0))
    cls_part = term1 + corr
    npos_part = jnp.sum(jnp.where(posmask, 1.0, 0.0))

    r = reg_ref[0]
    t0 = (m[1] / 0.1) - r[0]
    t1 = (m[2] / 0.1) - r[1]
    t2 = (jnp.log(m[3]) / 0.2) - r[2]
    t3 = (jnp.log(m[4]) / 0.2) - r[3]

    def smooth(d):
        d = jnp.abs(d)
        return jnp.where(d <= 1.0 / 9.0, 0.5 * 9.0 * d * d, d - 0.5 / 9.0)

    reg_anchor = smooth(t0) + smooth(t1) + smooth(t2) + smooth(t3)
    reg_part = jnp.sum(jnp.where(posmask, reg_anchor, 0.0))

    @pl.when(i == 0)
    def _():
        out_ref[...] = jnp.zeros_like(out_ref)

    ii = lax.broadcasted_iota(jnp.int32, (1, 3, 128), 1)
    upd = (
        jnp.where(ii == 0, cls_part, 0.0)
        + jnp.where(ii == 1, reg_part, 0.0)
        + jnp.where(ii == 2, npos_part, 0.0)
    )
    out_ref[...] += upd


def kernel(classifications, regressions, anchors, annotations, criterion,
           transcription, selected_indices, probs_sizes, pool_w, htr_gt_box):
    B, A, C = classifications.shape
    G = annotations.shape[1]
    nblk = (A + _AB - 1) // _AB
    apad = nblk * _AB
    gp = 64

    anchor = anchors[0].astype(jnp.float32)
    pad_col = jnp.array([0.0, 0.0, 1.0, 1.0], jnp.float32)
    anc_t = jnp.concatenate(
        [anchor.T, jnp.broadcast_to(pad_col[:, None], (4, apad - A))], axis=1
    ).reshape(4 * apad)
    b4 = annotations[:, :, :4].astype(jnp.float32)
    b4p = jnp.concatenate([b4, jnp.zeros((B, gp - G, 4), jnp.float32)], axis=1)
    barea = (b4p[:, :, 2] - b4p[:, :, 0]) * (b4p[:, :, 3] - b4p[:, :, 1])
    bb = jnp.concatenate([b4p, barea[:, :, None]], axis=2)
    bb = jnp.broadcast_to(bb[:, :, :, None], (B, gp, 5, _LANES))
    bb = bb.reshape(B * gp * 5 * _LANES)

    match = _sc_match(anc_t, b4p.reshape(B * gp * 4), bb,
                      apad=apad, gp=gp, g_real=G, n_batch=B)
    match4 = match.reshape(5, B * nblk, 1, _AB)

    reg_t = jnp.transpose(regressions[:, :, :4], (0, 2, 1))
    p0_all = jnp.pad(classifications[:, :, 0], ((0, 0), (0, apad - A)))
    p0_all = p0_all.reshape(B * nblk, 1, _AB)

    out = pl.pallas_call(
        functools.partial(_tc_body, a_real=A, nblk=nblk),
        grid=(B, nblk),
        in_specs=[
            pl.BlockSpec((1, _AB, C), lambda j, i: (j, i, 0)),
            pl.BlockSpec((1, 1, _AB), lambda j, i: (j * nblk + i, 0, 0)),
            pl.BlockSpec((1, 4, _AB), lambda j, i: (j, 0, i)),
            pl.BlockSpec((5, 1, 1, _AB), lambda j, i: (0, j * nblk + i, 0, 0)),
        ],
        out_specs=pl.BlockSpec((1, 3, 128), lambda j, i: (j, 0, 0)),
        out_shape=jax.ShapeDtypeStruct((B, 3, 128), jnp.float32),
    )(classifications, p0_all, reg_t, match4)

    sums = out[:, :, 0]
    cls_sum, reg_sum, num_pos = sums[:, 0], sums[:, 1], sums[:, 2]
    valid_any = jnp.any(annotations[:, :, 4] != -1.0, axis=1)
    hg = jnp.asarray(htr_gt_box, dtype=bool)
    cls_j = jnp.where(valid_any, cls_sum / jnp.maximum(num_pos, 1.0), 0.0)
    reg_j = jnp.where(valid_any & ((num_pos > 0) | hg),
                      reg_sum / (num_pos * 4.0), 0.0)
    return (jnp.mean(cls_j, keepdims=True), jnp.mean(reg_j, keepdims=True))
